# baseline (speedup 1.0000x reference)
import functools

import jax
import jax.numpy as jnp
from jax import lax
from jax.experimental import pallas as pl
from jax.experimental.pallas import tpu as pltpu
from jax.experimental.pallas import tpu_sc as plsc

NUM_CLUSTERS = 1024
DECAY = 0.999
EPSILON = 1e-06
BN = 2048

INTERP = False


def _pq_body(x_ref, means_ref, mn_ref, xn_ref, kcol_ref, ids_ref, newm_ref,
             cout_ref, sumx_ref):
    nb = pl.program_id(0)
    h = pl.program_id(1)
    nnb = pl.num_programs(0)
    K = NUM_CLUSTERS
    D = 64

    @pl.when(nb == 0)
    def _init():
        sumx_ref[pl.ds(h, 1)] = jnp.zeros_like(sumx_ref[pl.ds(h, 1)])

    xh = x_ref[0]
    m = means_ref[pl.ds(h, 1)][0]
    mn = mn_ref[pl.ds(h, 1)][0]
    xn = xn_ref[0, 0]
    kcol = kcol_ref[...]

    prod = lax.dot_general(m, xh, (((1,), (1,)), ((), ())),
                           preferred_element_type=jnp.float32)
    dists = -2.0 * prod + xn + mn

    dmin = jnp.min(dists, axis=0, keepdims=True)
    eqmask = dists == dmin
    ids_f = jnp.min(jnp.where(eqmask, kcol, float(K)), axis=0,
                    keepdims=True)
    ids_ref[0, 0] = ids_f.astype(jnp.int32)

    ohT = eqmask.astype(jnp.float32)
    xh65 = jnp.concatenate(
        [xh, jnp.ones((BN, 1), jnp.float32)], axis=1)
    sumx_ref[pl.ds(h, 1)] += lax.dot_general(
        ohT, xh65, (((1,), (0,)), ((), ())),
        preferred_element_type=jnp.float32)[None]

    @pl.when(nb == nnb - 1)
    def _fin():
        acc = sumx_ref[pl.ds(h, 1)][0]
        pad = jnp.zeros((K, 128 - D), jnp.float32)
        newm_ref[pl.ds(h, 1)] = jnp.concatenate([acc[:, :-1], pad],
                                                axis=1)[None]
        cout_ref[pl.ds(h, 1)] = acc[:, -1:][None]


def _sc_ema_body(sumx_ref, cnt_ref, means_ref, out_ref,
                 xacc_ref, cbuf_ref, mbuf_ref, obuf_ref, dsem):
    c = lax.axis_index("c")
    s = lax.axis_index("s")
    w = c * 16 + s

    for cb in range(4):
        base = w * 512 + cb * 128
        pltpu.async_copy(sumx_ref.at[pl.ds(base, 128)], xacc_ref, dsem).wait()
        pltpu.async_copy(cnt_ref.at[pl.ds(base, 128)], cbuf_ref, dsem).wait()
        pltpu.async_copy(means_ref.at[pl.ds(base, 128)], mbuf_ref, dsem).wait()

        def _rbody(g, carry):
            denv = EPSILON + cbuf_ref[pl.ds(g * 16, 16)]
            for i in range(16):
                r = g * 16 + i
                den = denv[i]
                for jj in range(8):
                    sx = xacc_ref[r, pl.ds(jj * 16, 16)]
                    mv = mbuf_ref[r, pl.ds(jj * 16, 16)]
                    obuf_ref[r, pl.ds(jj * 16, 16)] = (
                        (1.0 - DECAY) * (sx / den) + DECAY * mv)
            return carry
        lax.fori_loop(0, 8, _rbody, 0)
        pltpu.async_copy(obuf_ref, out_ref.at[pl.ds(base, 128)], dsem).wait()


def kernel(x, means):
    B, L, H, D = x.shape
    K = means.shape[1]
    N = B * L
    nnb = N // BN

    x_r = jnp.transpose(x.reshape(N, H, D), (1, 0, 2))
    mn = jnp.sum(means * means, axis=2)[:, :, None]
    xn = jnp.transpose(jnp.sum(x.reshape(N, H, D) * x.reshape(N, H, D),
                               axis=2), (1, 0))
    xn = xn.reshape(H, nnb, 1, BN)
    kcol = lax.broadcasted_iota(jnp.float32, (K, 1), 0)

    ids4, sumx64, cnts = pl.pallas_call(
        _pq_body,
        grid=(nnb, H),
        in_specs=[
            pl.BlockSpec((1, BN, D), lambda nb, h: (h, nb, 0)),
            pl.BlockSpec((H, K, D), lambda nb, h: (0, 0, 0)),
            pl.BlockSpec((H, K, 1), lambda nb, h: (0, 0, 0)),
            pl.BlockSpec((1, 1, 1, BN), lambda nb, h: (h, nb, 0, 0)),
            pl.BlockSpec((K, 1), lambda nb, h: (0, 0)),
        ],
        out_specs=[
            pl.BlockSpec((1, 1, 1, BN), lambda nb, h: (nb, h, 0, 0)),
            pl.BlockSpec((H, K, 128), lambda nb, h: (0, 0, 0)),
            pl.BlockSpec((H, K, 1), lambda nb, h: (0, 0, 0)),
        ],
        out_shape=[
            jax.ShapeDtypeStruct((nnb, H, 1, BN), jnp.int32),
            jax.ShapeDtypeStruct((H, K, 128), jnp.float32),
            jax.ShapeDtypeStruct((H, K, 1), jnp.float32),
        ],
        scratch_shapes=[
            pltpu.VMEM((H, K, D + 1), jnp.float32),
        ],
        compiler_params=pltpu.CompilerParams(
            dimension_semantics=("arbitrary", "arbitrary"),
        ),
        interpret=INTERP,
    )(x_r, means, mn, xn, kcol)

    mesh = plsc.VectorSubcoreMesh(core_axis_name="c", subcore_axis_name="s")
    sck = functools.partial(
        pl.kernel,
        out_type=jax.ShapeDtypeStruct((H * K, 128), jnp.float32),
        mesh=mesh,
        scratch_types=[
            pltpu.VMEM((128, 128), jnp.float32),
            pltpu.VMEM((128,), jnp.float32),
            pltpu.VMEM((128, 128), jnp.float32),
            pltpu.VMEM((128, 128), jnp.float32),
            pltpu.SemaphoreType.DMA,
        ],
    )(_sc_ema_body)
    mpad = jnp.pad(means.reshape(H * K, D), ((0, 0), (0, 128 - D)))
    nm128 = sck(sumx64.reshape(H * K, 128), cnts.reshape(H * K), mpad)
    new_means = nm128[:, :D].reshape(H, K, D)

    cluster_ids = jnp.transpose(ids4.reshape(nnb, H, BN), (0, 2, 1))
    cluster_ids = cluster_ids.reshape(B, L, H)
    return cluster_ids, new_means


# speedup vs baseline: 1.3389x; 1.3389x over previous
import jax
import jax.numpy as jnp
from jax import lax
from jax.experimental import pallas as pl
from jax.experimental.pallas import tpu as pltpu

NUM_CLUSTERS = 1024
DECAY = 0.999
EPSILON = 1e-06
BN = 2048


def _pq_body(x_ref, means_ref, mn_ref, xn_ref, kcol_ref, ids_ref, newm_ref,
             sumx_ref):
    nb = pl.program_id(0)
    h = pl.program_id(1)
    nnb = pl.num_programs(0)
    K = NUM_CLUSTERS

    @pl.when(nb == 0)
    def _init():
        sumx_ref[pl.ds(h, 1)] = jnp.zeros_like(sumx_ref[pl.ds(h, 1)])

    xh = x_ref[0]
    m = means_ref[pl.ds(h, 1)][0]
    mn = mn_ref[pl.ds(h, 1)][0]
    xn = xn_ref[0, 0]
    kcol = kcol_ref[...]

    prod = lax.dot_general(m, xh, (((1,), (1,)), ((), ())),
                           preferred_element_type=jnp.float32)
    dists = -2.0 * prod + xn + mn

    dmin = jnp.min(dists, axis=0, keepdims=True)
    eqmask = dists == dmin
    ids_f = jnp.min(jnp.where(eqmask, kcol, float(K)), axis=0,
                    keepdims=True)
    ids_ref[0, 0] = ids_f.astype(jnp.int32)

    ohT = eqmask.astype(jnp.float32)
    xh65 = jnp.concatenate(
        [xh, jnp.ones((BN, 1), jnp.float32)], axis=1)
    sumx_ref[pl.ds(h, 1)] += lax.dot_general(
        ohT, xh65, (((1,), (0,)), ((), ())),
        preferred_element_type=jnp.float32)[None]

    @pl.when(nb == nnb - 1)
    def _fin():
        acc = sumx_ref[pl.ds(h, 1)][0]
        cnt = acc[:, -1:]
        meansx = acc[:, :-1] / (EPSILON + cnt)
        newm_ref[pl.ds(h, 1)] = ((1.0 - DECAY) * meansx + DECAY * m)[None]


def kernel(x, means):
    B, L, H, D = x.shape
    K = means.shape[1]
    N = B * L
    nnb = N // BN

    x_r = jnp.transpose(x.reshape(N, H, D), (1, 0, 2))
    mn = jnp.sum(means * means, axis=2)[:, :, None]
    xn = jnp.transpose(jnp.sum(x.reshape(N, H, D) * x.reshape(N, H, D),
                               axis=2), (1, 0))
    xn = xn.reshape(H, nnb, 1, BN)
    kcol = lax.broadcasted_iota(jnp.float32, (K, 1), 0)

    ids4, new_means = pl.pallas_call(
        _pq_body,
        grid=(nnb, H),
        in_specs=[
            pl.BlockSpec((1, BN, D), lambda nb, h: (h, nb, 0)),
            pl.BlockSpec((H, K, D), lambda nb, h: (0, 0, 0)),
            pl.BlockSpec((H, K, 1), lambda nb, h: (0, 0, 0)),
            pl.BlockSpec((1, 1, 1, BN), lambda nb, h: (h, nb, 0, 0)),
            pl.BlockSpec((K, 1), lambda nb, h: (0, 0)),
        ],
        out_specs=[
            pl.BlockSpec((1, 1, 1, BN), lambda nb, h: (nb, h, 0, 0)),
            pl.BlockSpec((H, K, D), lambda nb, h: (0, 0, 0)),
        ],
        out_shape=[
            jax.ShapeDtypeStruct((nnb, H, 1, BN), jnp.int32),
            jax.ShapeDtypeStruct((H, K, D), jnp.float32),
        ],
        scratch_shapes=[
            pltpu.VMEM((H, K, D + 1), jnp.float32),
        ],
        compiler_params=pltpu.CompilerParams(
            dimension_semantics=("arbitrary", "arbitrary"),
        ),
    )(x_r, means, mn, xn, kcol)

    cluster_ids = jnp.transpose(ids4.reshape(nnb, H, BN), (0, 2, 1))
    cluster_ids = cluster_ids.reshape(B, L, H)
    return cluster_ids, new_means
